# Initial kernel scaffold; baseline (speedup 1.0000x reference)
import jax
import jax.numpy as jnp
from jax.experimental import pallas as pl


def kernel(x, edge_index, W1, b1, W2, b2):
    raise NotImplementedError("write your pallas kernel here")


# speedup vs baseline: 7.9639x; 7.9639x over previous
import functools

import jax
import jax.numpy as jnp
from jax import lax
from jax.experimental import pallas as pl
from jax.experimental.pallas import tpu as pltpu
from jax.experimental.pallas import tpu_sc as plsc

N = 10000
E = 320000
D_IN = 128
D_H = 128
D_OUT = 64

NC = 2
NS = 16
NW = NC * NS

SUB = 128
KROWS = 8
ROWS_PER_TILE = 80
OUTER = ROWS_PER_TILE // KROWS
EPAD = NW * ROWS_PER_TILE * SUB
EROWS = EPAD // SUB
NPAD = EPAD // NW
DEG_W = 128

@functools.cache
def _get_deg_pass():
    mesh = plsc.VectorSubcoreMesh(core_axis_name="c", subcore_axis_name="s")

    @functools.partial(
        pl.kernel,
        out_type=jax.ShapeDtypeStruct((NC, NPAD, DEG_W), jnp.float32),
        mesh=mesh,
        scratch_types=[
            pltpu.VMEM((KROWS, SUB), jnp.int32),
            pltpu.VMEM((SUB, DEG_W), jnp.float32),
            pltpu.VMEM_SHARED((NPAD, DEG_W), jnp.float32),
        ],
    )
    def _deg_pass(dst_hbm, zeros_hbm, ones_hbm, out_hbm, dst_v, ones_v, acc):
        c = lax.axis_index("c")
        s = lax.axis_index("s")
        wid = c * NS + s
        pltpu.sync_copy(ones_hbm, ones_v)
        rz = NPAD // NS
        pltpu.sync_copy(zeros_hbm.at[pl.ds(s * rz, rz)],
                        acc.at[pl.ds(s * rz, rz)])
        plsc.subcore_barrier()
        row0 = wid * ROWS_PER_TILE

        @pl.loop(0, OUTER)
        def _(o):
            pltpu.sync_copy(dst_hbm.at[pl.ds(row0 + o * KROWS, KROWS)], dst_v)
            for k in range(KROWS):
                pltpu.sync_copy(ones_v, acc.at[dst_v.at[k]], add=True)

        plsc.subcore_barrier()
        pltpu.sync_copy(acc.at[pl.ds(s * rz, rz)],
                        out_hbm.at[c, pl.ds(s * rz, rz)])

    return _deg_pass


@functools.cache
def _make_edge_pass(D):
    mesh = plsc.VectorSubcoreMesh(core_axis_name="c", subcore_axis_name="s")

    @functools.partial(
        pl.kernel,
        out_type=jax.ShapeDtypeStruct((NC, NPAD, D), jnp.float32),
        mesh=mesh,
        scratch_types=[
            pltpu.VMEM((KROWS, SUB), jnp.int32),
            pltpu.VMEM((KROWS, SUB), jnp.int32),
            pltpu.VMEM((SUB, D), jnp.float32),
            pltpu.VMEM_SHARED((NPAD, D), jnp.float32),
        ],
    )
    def edge_pass(hs_hbm, src_hbm, dst_hbm, zeros_hbm, out_hbm,
                  src_v, dst_v, rows_v, acc):
        c = lax.axis_index("c")
        s = lax.axis_index("s")
        wid = c * NS + s
        rz = NPAD // NS
        pltpu.sync_copy(zeros_hbm.at[pl.ds(s * rz, rz)],
                        acc.at[pl.ds(s * rz, rz)])
        plsc.subcore_barrier()
        row0 = wid * ROWS_PER_TILE

        @pl.loop(0, OUTER)
        def _(o):
            pltpu.sync_copy(src_hbm.at[pl.ds(row0 + o * KROWS, KROWS)], src_v)
            pltpu.sync_copy(dst_hbm.at[pl.ds(row0 + o * KROWS, KROWS)], dst_v)
            for k in range(KROWS):
                pltpu.sync_copy(hs_hbm.at[src_v.at[k]], rows_v)
                pltpu.sync_copy(rows_v, acc.at[dst_v.at[k]], add=True)

        plsc.subcore_barrier()
        pltpu.sync_copy(acc.at[pl.ds(s * rz, rz)],
                        out_hbm.at[c, pl.ds(s * rz, rz)])

    return edge_pass


_R = 1000


def _tc_hs1_body(x_ref, w_ref, degp_ref, o_ref):
    p = degp_ref[0, :, 0:1] + degp_ref[1, :, 0:1]
    dinv = lax.rsqrt(1.0 + p)
    o_ref[...] = dinv * jnp.dot(x_ref[...], w_ref[...],
                                preferred_element_type=jnp.float32)


def _tc_mid_body(aggp_ref, hs_ref, degp_ref, b_ref, w_ref, o_ref):
    p = degp_ref[0, :, 0:1] + degp_ref[1, :, 0:1]
    dinv = lax.rsqrt(1.0 + p)
    t = dinv * (aggp_ref[0] + aggp_ref[1] + hs_ref[...]) + b_ref[...]
    out1 = jnp.maximum(t, 0.0)
    o_ref[...] = dinv * jnp.dot(out1, w_ref[...],
                                preferred_element_type=jnp.float32)


def _tc_fin_body(aggp_ref, hs_ref, degp_ref, b_ref, o_ref):
    p = degp_ref[0, :, 0:1] + degp_ref[1, :, 0:1]
    dinv = lax.rsqrt(1.0 + p)
    agg = aggp_ref[0, :, :D_OUT] + aggp_ref[1, :, :D_OUT]
    z = dinv * (agg + hs_ref[...]) + b_ref[...]
    m = jnp.max(z, axis=1, keepdims=True)
    e = jnp.exp(z - m)
    lse = jnp.log(jnp.sum(e, axis=1, keepdims=True)) + m
    o_ref[...] = z - lse


def _tc_hs1(x, W1, degp):
    return pl.pallas_call(
        _tc_hs1_body,
        grid=(N // _R,),
        in_specs=[
            pl.BlockSpec((_R, D_IN), lambda i: (i, 0)),
            pl.BlockSpec((D_IN, D_H), lambda i: (0, 0)),
            pl.BlockSpec((NC, _R, DEG_W), lambda i: (0, i, 0)),
        ],
        out_specs=pl.BlockSpec((_R, D_H), lambda i: (i, 0)),
        out_shape=jax.ShapeDtypeStruct((N, D_H), jnp.float32),
    )(x, W1, degp)


def _tc_mid(agg1p, hs1, degp, b1, W2):
    return pl.pallas_call(
        _tc_mid_body,
        grid=(N // _R,),
        in_specs=[
            pl.BlockSpec((NC, _R, D_H), lambda i: (0, i, 0)),
            pl.BlockSpec((_R, D_H), lambda i: (i, 0)),
            pl.BlockSpec((NC, _R, DEG_W), lambda i: (0, i, 0)),
            pl.BlockSpec((1, D_H), lambda i: (0, 0)),
            pl.BlockSpec((D_H, D_OUT), lambda i: (0, 0)),
        ],
        out_specs=pl.BlockSpec((_R, D_OUT), lambda i: (i, 0)),
        out_shape=jax.ShapeDtypeStruct((N, D_OUT), jnp.float32),
    )(agg1p, hs1, degp, b1, W2)


def _tc_fin(agg2p, hs2, degp, b2):
    return pl.pallas_call(
        _tc_fin_body,
        grid=(N // _R,),
        in_specs=[
            pl.BlockSpec((NC, _R, D_H), lambda i: (0, i, 0)),
            pl.BlockSpec((_R, D_OUT), lambda i: (i, 0)),
            pl.BlockSpec((NC, _R, DEG_W), lambda i: (0, i, 0)),
            pl.BlockSpec((1, D_OUT), lambda i: (0, 0)),
        ],
        out_specs=pl.BlockSpec((_R, D_OUT), lambda i: (i, 0)),
        out_shape=jax.ShapeDtypeStruct((N, D_OUT), jnp.float32),
    )(agg2p, hs2, degp, b2)


def kernel(x, edge_index, W1, b1, W2, b2):
    src = edge_index[0].astype(jnp.int32)
    dst = edge_index[1].astype(jnp.int32)
    pad = jnp.full((EPAD - E,), N, jnp.int32)
    srcp = jnp.concatenate([src, pad]).reshape(EROWS, SUB)
    dstp = jnp.concatenate([dst, pad]).reshape(EROWS, SUB)

    zeros_h = jnp.zeros((NPAD, D_H), jnp.float32)
    ones = jnp.ones((SUB, DEG_W), jnp.float32)

    degp = _get_deg_pass()(dstp, zeros_h, ones)

    hs1 = _tc_hs1(x, W1, degp)
    hs1p = jnp.concatenate(
        [hs1, jnp.zeros((NPAD - N, D_H), jnp.float32)], axis=0)
    agg1p = _make_edge_pass(D_H)(hs1p, srcp, dstp, zeros_h)

    hs2 = _tc_mid(agg1p, hs1, degp, b1.reshape(1, D_H), W2)
    hs2p = jnp.zeros((NPAD, D_H), jnp.float32).at[:N, :D_OUT].set(hs2)
    agg2p = _make_edge_pass(D_H)(hs2p, srcp, dstp, zeros_h)

    return _tc_fin(agg2p, hs2, degp, b2.reshape(1, D_OUT))


# speedup vs baseline: 9.0789x; 1.1400x over previous
import functools

import jax
import jax.numpy as jnp
from jax import lax
from jax.experimental import pallas as pl
from jax.experimental.pallas import tpu as pltpu
from jax.experimental.pallas import tpu_sc as plsc

N = 10000
E = 320000
D_IN = 128
D_H = 128
D_OUT = 64

NC = 2
NS = 16
NW = NC * NS

SUB = 128
KROWS = 8
ROWS_PER_TILE = 80
OUTER = ROWS_PER_TILE // KROWS
EPAD = NW * ROWS_PER_TILE * SUB
EROWS = EPAD // SUB
NPAD = EPAD // NW
L = 16

@functools.cache
def _get_deg_pass():
    mesh = plsc.VectorSubcoreMesh(core_axis_name="c", subcore_axis_name="s")
    rz = NPAD // NS

    @functools.partial(
        pl.kernel,
        out_type=jax.ShapeDtypeStruct((NC, NPAD), jnp.float32),
        mesh=mesh,
        scratch_types=[
            pltpu.VMEM((ROWS_PER_TILE, SUB), jnp.int32),
            pltpu.VMEM((NPAD,), jnp.float32),
            pltpu.VMEM((NS, rz), jnp.float32),
            pltpu.VMEM((rz,), jnp.float32),
            pltpu.VMEM_SHARED((NS, NPAD), jnp.float32),
        ],
        compiler_params=pltpu.CompilerParams(needs_layout_passes=False),
    )
    def _deg_pass(dst_hbm, out_hbm, dst_v, acc_v, red_v, out_v, sh):
        c = lax.axis_index("c")
        s = lax.axis_index("s")
        wid = c * NS + s
        row0 = wid * ROWS_PER_TILE
        pltpu.sync_copy(dst_hbm.at[pl.ds(row0, ROWS_PER_TILE)], dst_v)

        zeros = jnp.zeros((L,), jnp.float32)

        @pl.loop(0, NPAD, step=L)
        def _(j):
            acc_v[pl.ds(j, L)] = zeros

        ones = jnp.ones((L,), jnp.float32)

        @pl.loop(0, ROWS_PER_TILE)
        def _(r):
            for k in range(SUB // L):
                idx = dst_v[r, pl.ds(k * L, L)]
                plsc.addupdate_scatter(acc_v, [idx], ones)

        pltpu.sync_copy(acc_v, sh.at[s])
        plsc.subcore_barrier()
        for r in range(NS):
            pltpu.sync_copy(sh.at[r, pl.ds(s * rz, rz)], red_v.at[r])

        @pl.loop(0, rz, step=L)
        def _(j):
            v = red_v[0, pl.ds(j, L)]
            for r in range(1, NS):
                v = v + red_v[r, pl.ds(j, L)]
            out_v[pl.ds(j, L)] = v

        pltpu.sync_copy(out_v, out_hbm.at[c, pl.ds(s * rz, rz)])

    return _deg_pass


NBUF = 2
HALF = ROWS_PER_TILE // 2


@functools.cache
def _make_edge_pass(D):
    mesh = plsc.VectorSubcoreMesh(core_axis_name="c", subcore_axis_name="s")

    @functools.partial(
        pl.kernel,
        out_type=jax.ShapeDtypeStruct((NC, NPAD, D), jnp.float32),
        mesh=mesh,
        scratch_types=[
            pltpu.VMEM((HALF, SUB), jnp.int32),
            pltpu.VMEM((HALF, SUB), jnp.int32),
        ]
        + [pltpu.VMEM((SUB, D), jnp.float32) for _ in range(NBUF)]
        + [pltpu.SemaphoreType.DMA for _ in range(2 * NBUF)]
        + [pltpu.VMEM_SHARED((NPAD, D), jnp.float32)],
    )
    def edge_pass(hs_hbm, src_hbm, dst_hbm, zeros_hbm, out_hbm,
                  src_v, dst_v, *rest):
        bufs = rest[:NBUF]
        sg = rest[NBUF:2 * NBUF]
        ss = rest[2 * NBUF:3 * NBUF]
        acc = rest[3 * NBUF]
        c = lax.axis_index("c")
        s = lax.axis_index("s")
        wid = c * NS + s
        rz = NPAD // NS
        pltpu.sync_copy(zeros_hbm.at[pl.ds(s * rz, rz)],
                        acc.at[pl.ds(s * rz, rz)])
        plsc.subcore_barrier()
        row0 = wid * ROWS_PER_TILE

        @pl.loop(0, 2)
        def _(h):
            r0 = row0 + h * HALF
            pltpu.sync_copy(src_hbm.at[pl.ds(r0, HALF)], src_v)
            pltpu.sync_copy(dst_hbm.at[pl.ds(r0, HALF)], dst_v)
            for b in range(NBUF):
                pltpu.async_copy(hs_hbm.at[src_v.at[b]], bufs[b], sg[b])

            @pl.loop(0, HALF - NBUF, step=NBUF)
            def _(j):
                for b in range(NBUF):
                    pltpu.make_async_copy(
                        hs_hbm.at[src_v.at[j + b]], bufs[b], sg[b]).wait()
                    pltpu.async_copy(
                        bufs[b], acc.at[dst_v.at[j + b]], ss[b], add=True)
                for b in range(NBUF):
                    pltpu.make_async_copy(
                        bufs[b], acc.at[dst_v.at[j + b]], ss[b]).wait()
                    pltpu.async_copy(
                        hs_hbm.at[src_v.at[j + NBUF + b]], bufs[b], sg[b])

            j0 = HALF - NBUF
            for b in range(NBUF):
                pltpu.make_async_copy(
                    hs_hbm.at[src_v.at[j0 + b]], bufs[b], sg[b]).wait()
                pltpu.async_copy(
                    bufs[b], acc.at[dst_v.at[j0 + b]], ss[b], add=True)
            for b in range(NBUF):
                pltpu.make_async_copy(
                    bufs[b], acc.at[dst_v.at[j0 + b]], ss[b]).wait()

        plsc.subcore_barrier()
        pltpu.sync_copy(acc.at[pl.ds(s * rz, rz)],
                        out_hbm.at[c, pl.ds(s * rz, rz)])

    return edge_pass


_R = 1000


def _dinv_col(degp_ref):
    p = degp_ref[0] + degp_ref[1]
    return lax.rsqrt(1.0 + p)


def _tc_hs1_body(x_ref, w_ref, degp_ref, o_ref):
    dinv = _dinv_col(degp_ref)
    o_ref[...] = dinv * jnp.dot(x_ref[...], w_ref[...],
                                preferred_element_type=jnp.float32)


def _tc_mid_body(aggp_ref, hs_ref, degp_ref, b_ref, w_ref, o_ref):
    dinv = _dinv_col(degp_ref)
    t = dinv * (aggp_ref[0] + aggp_ref[1] + hs_ref[...]) + b_ref[...]
    out1 = jnp.maximum(t, 0.0)
    o_ref[...] = dinv * jnp.dot(out1, w_ref[...],
                                preferred_element_type=jnp.float32)


def _tc_fin_body(aggp_ref, hs_ref, degp_ref, b_ref, o_ref):
    dinv = _dinv_col(degp_ref)
    agg = aggp_ref[0, :, :D_OUT] + aggp_ref[1, :, :D_OUT]
    z = dinv * (agg + hs_ref[...]) + b_ref[...]
    m = jnp.max(z, axis=1, keepdims=True)
    e = jnp.exp(z - m)
    lse = jnp.log(jnp.sum(e, axis=1, keepdims=True)) + m
    o_ref[...] = z - lse


def _tc_hs1(x, W1, degp):
    return pl.pallas_call(
        _tc_hs1_body,
        grid=(N // _R,),
        in_specs=[
            pl.BlockSpec((_R, D_IN), lambda i: (i, 0)),
            pl.BlockSpec((D_IN, D_H), lambda i: (0, 0)),
            pl.BlockSpec((NC, _R, 1), lambda i: (0, i, 0)),
        ],
        out_specs=pl.BlockSpec((_R, D_H), lambda i: (i, 0)),
        out_shape=jax.ShapeDtypeStruct((N, D_H), jnp.float32),
    )(x, W1, degp)


def _tc_mid(agg1p, hs1, degp, b1, W2):
    return pl.pallas_call(
        _tc_mid_body,
        grid=(N // _R,),
        in_specs=[
            pl.BlockSpec((NC, _R, D_H), lambda i: (0, i, 0)),
            pl.BlockSpec((_R, D_H), lambda i: (i, 0)),
            pl.BlockSpec((NC, _R, 1), lambda i: (0, i, 0)),
            pl.BlockSpec((1, D_H), lambda i: (0, 0)),
            pl.BlockSpec((D_H, D_OUT), lambda i: (0, 0)),
        ],
        out_specs=pl.BlockSpec((_R, D_OUT), lambda i: (i, 0)),
        out_shape=jax.ShapeDtypeStruct((N, D_OUT), jnp.float32),
    )(agg1p, hs1, degp, b1, W2)


def _tc_fin(agg2p, hs2, degp, b2):
    return pl.pallas_call(
        _tc_fin_body,
        grid=(N // _R,),
        in_specs=[
            pl.BlockSpec((NC, _R, D_H), lambda i: (0, i, 0)),
            pl.BlockSpec((_R, D_OUT), lambda i: (i, 0)),
            pl.BlockSpec((NC, _R, 1), lambda i: (0, i, 0)),
            pl.BlockSpec((1, D_OUT), lambda i: (0, 0)),
        ],
        out_specs=pl.BlockSpec((_R, D_OUT), lambda i: (i, 0)),
        out_shape=jax.ShapeDtypeStruct((N, D_OUT), jnp.float32),
    )(agg2p, hs2, degp, b2)


def kernel(x, edge_index, W1, b1, W2, b2):
    src = edge_index[0].astype(jnp.int32)
    dst = edge_index[1].astype(jnp.int32)
    pad = jnp.full((EPAD - E,), N, jnp.int32)
    srcp = jnp.concatenate([src, pad]).reshape(EROWS, SUB)
    dstp = jnp.concatenate([dst, pad]).reshape(EROWS, SUB)

    zeros_h = jnp.zeros((NPAD, D_H), jnp.float32)

    degp = _get_deg_pass()(dstp)[:, :, None]

    hs1 = _tc_hs1(x, W1, degp)
    hs1p = jnp.concatenate(
        [hs1, jnp.zeros((NPAD - N, D_H), jnp.float32)], axis=0)
    agg1p = _make_edge_pass(D_H)(hs1p, srcp, dstp, zeros_h)

    hs2 = _tc_mid(agg1p, hs1, degp, b1.reshape(1, D_H), W2)
    hs2p = jnp.zeros((NPAD, D_H), jnp.float32).at[:N, :D_OUT].set(hs2)
    agg2p = _make_edge_pass(D_H)(hs2p, srcp, dstp, zeros_h)

    return _tc_fin(agg2p, hs2, degp, b2.reshape(1, D_OUT))
